# baseline (speedup 1.0000x reference)
import functools

import jax
import jax.numpy as jnp
from jax import lax
from jax.experimental import pallas as pl
from jax.experimental.pallas import tpu as pltpu
from jax.experimental.pallas import tpu_sc as plsc

K = 40
B, N = 4, 1024
NC, NS, L = 2, 16, 16
NW = NC * NS
P = (B * N) // NW
PC = 8
RK = PC * K
NCH = P // PC
CH = 32
NB = N // CH


def _pre_body(h_ref, xx_ref, pd_ref, tk_ref):
    h = h_ref[0]
    g = lax.dot_general(h, h, (((1,), (1,)), ((), ())),
                        preferred_element_type=jnp.float32)
    pd = 2.0 * g - xx_ref[0]
    pd_ref[0] = pd
    s = lax.bitcast_convert_type(pd, jnp.int32)
    key = jnp.where(s < 0, s ^ jnp.int32(0x7FFFFFFF), s)

    def bit_step(i, t):
        bit = jnp.int32(1) << (jnp.int32(30) - i)
        cand = t + bit
        cnt = jnp.sum((key >= cand).astype(jnp.float32), axis=1, keepdims=True)
        return jnp.where(cnt >= float(K), cand, t)

    cnt0 = jnp.sum((key >= 0).astype(jnp.float32), axis=1, keepdims=True)
    t0 = jnp.where(cnt0 >= float(K), jnp.int32(0), jnp.int32(-2147483648))
    t = lax.fori_loop(0, 31, bit_step, t0)
    tk_ref[0] = jnp.broadcast_to(t, (N, L))


def _pre(h, xx):
    C = h.shape[2]
    return pl.pallas_call(
        _pre_body,
        grid=(B,),
        in_specs=[
            pl.BlockSpec((1, N, C), lambda b: (b, 0, 0)),
            pl.BlockSpec((1, 1, N), lambda b: (b, 0, 0)),
        ],
        out_specs=[
            pl.BlockSpec((1, N, N), lambda b: (b, 0, 0)),
            pl.BlockSpec((1, N, L), lambda b: (b, 0, 0)),
        ],
        out_shape=[
            jax.ShapeDtypeStruct((B, N, N), jnp.float32),
            jax.ShapeDtypeStruct((B, N, L), jnp.int32),
        ],
    )(h, xx)


def _make_gather(C):
    mesh = plsc.VectorSubcoreMesh(core_axis_name="c", subcore_axis_name="s",
                                  num_cores=NC, num_subcores=NS)

    @functools.partial(
        pl.kernel,
        out_type=jax.ShapeDtypeStruct((B * N * K, C), jnp.float32),
        mesh=mesh,
        compiler_params=pltpu.CompilerParams(use_tc_tiling_on_sc=False,
                                             needs_layout_passes=False),
        scratch_types=[
            pltpu.VMEM((P * K,), jnp.int32),
            pltpu.VMEM((RK, C), jnp.float32),
            pltpu.VMEM((RK, C), jnp.float32),
            pltpu.VMEM((N,), jnp.float32),
            pltpu.VMEM((N,), jnp.float32),
            pltpu.VMEM((P, L), jnp.int32),
            pltpu.SemaphoreType.DMA,
            pltpu.SemaphoreType.DMA,
            pltpu.SemaphoreType.DMA,
            pltpu.SemaphoreType.DMA,
        ],
    )
    def gather_kernel(h_hbm, pd_hbm, tk_hbm, nbr_hbm,
                      idx_v, b0, b1, r0, r1, tk_v,
                      sem0, sem1, rs0, rs1):
        wid = lax.axis_index("s") * NC + lax.axis_index("c")
        base = wid * P * K
        pbase = wid * P
        bn0 = (pbase // N) * N
        pltpu.sync_copy(tk_hbm.at[pl.ds(pbase, P)], tk_v)

        rbufs = (r0, r1)
        rsems = (rs0, rs1)

        def rfire(p, slot):
            pltpu.async_copy(pd_hbm.at[pl.ds((pbase + p) * N, N)],
                             rbufs[slot], rsems[slot])

        def rwait(slot):
            pltpu.make_async_copy(pd_hbm.at[pl.ds(0, N)],
                                  rbufs[slot], rsems[slot]).wait()

        iota16 = lax.iota(jnp.int32, L)

        def select(p, slot):
            row = rbufs[slot]
            tvec = tk_v[p]

            def jstep(j, run):
                v = row[pl.ds(j * L, L)]
                s = plsc.bitcast(v, jnp.int32)
                keyv = jnp.where(s < 0, s ^ jnp.int32(0x7FFFFFFF), s)
                ge = keyv >= tvec
                gei = jnp.where(ge, jnp.int32(1), jnp.int32(0))
                incl = plsc.cumsum(gei)
                pos = run + (incl - gei)
                okm = jnp.logical_and(ge, pos < K)
                val = iota16 + (bn0 + j * L)
                plsc.store_scatter(idx_v, (pos + p * K,), val, mask=okm)
                return run + plsc.all_reduce_population_count(ge)

            lax.fori_loop(0, N // L, jstep, jnp.zeros((L,), jnp.int32))

        rfire(0, 0)

        def sel_step(g, _):
            p0 = 2 * g
            p1 = 2 * g + 1
            rfire(p1, 1)
            rwait(0)
            select(p0, 0)

            @pl.when(p1 + 1 < P)
            def _():
                rfire(p1 + 1, 0)

            rwait(1)
            select(p1, 1)
            return 0

        lax.fori_loop(0, P // 2, sel_step, 0)

        bufs = (b0, b1)
        sems = (sem0, sem1)

        def fire(c, slot):
            pltpu.async_copy(h_hbm.at[idx_v.at[pl.ds(c * RK, RK)]],
                             bufs[slot], sems[slot])

        def wait(slot):
            pltpu.make_async_copy(h_hbm.at[idx_v.at[pl.ds(0, RK)]],
                                  bufs[slot], sems[slot]).wait()

        def flush(c, slot):
            pltpu.sync_copy(bufs[slot], nbr_hbm.at[pl.ds(base + c * RK, RK)])

        fire(0, 0)

        def step(g, _):
            c0 = 2 * g
            c1 = 2 * g + 1
            fire(c1, 1)
            wait(0)
            flush(c0, 0)

            @pl.when(c1 + 1 < NCH)
            def _():
                fire(c1 + 1, 0)

            wait(1)
            flush(c1, 1)
            return 0

        lax.fori_loop(0, NCH // 2, step, 0)

    return gather_kernel


_GATHER_CACHE = {}


def _gather(C):
    if C not in _GATHER_CACHE:
        _GATHER_CACHE[C] = _make_gather(C)
    return _GATHER_CACHE[C]


def _conv_body(nbr_ref, h_ref, w1t_ref, w2t_ref, ymax_ref, ssum_ref, ssq_ref):
    b = pl.program_id(0)
    n = pl.program_id(1)
    ctr = h_ref[0, 0]
    nbr = nbr_ref[0, 0]
    C = ctr.shape[1]
    diff = (nbr.reshape(CH, K, C) - ctr[:, None, :]).reshape(CH * K, C)
    yd = jnp.dot(diff, w1t_ref[...], preferred_element_type=jnp.float32)
    yc = jnp.dot(ctr, w2t_ref[...], preferred_element_type=jnp.float32)
    O = yd.shape[1]
    y = yd.reshape(CH, K, O) + yc[:, None, :]
    ymax_ref[0, 0] = jnp.max(y, axis=1)
    y2 = y.reshape(CH * K, O)
    ps = jnp.sum(y2, axis=0, keepdims=True)
    pq = jnp.sum(y2 * y2, axis=0, keepdims=True)

    @pl.when(jnp.logical_and(b == 0, n == 0))
    def _():
        ssum_ref[...] = ps
        ssq_ref[...] = pq

    @pl.when(jnp.logical_or(b > 0, n > 0))
    def _():
        ssum_ref[...] = ssum_ref[...] + ps
        ssq_ref[...] = ssq_ref[...] + pq


def _conv(nbr4, h4, w1t, w2t):
    C = w1t.shape[0]
    O = w1t.shape[1]
    return pl.pallas_call(
        _conv_body,
        grid=(B, NB),
        in_specs=[
            pl.BlockSpec((1, 1, CH * K, C), lambda b, n: (b, n, 0, 0)),
            pl.BlockSpec((1, 1, CH, C), lambda b, n: (b, n, 0, 0)),
            pl.BlockSpec((C, O), lambda b, n: (0, 0)),
            pl.BlockSpec((C, O), lambda b, n: (0, 0)),
        ],
        out_specs=[
            pl.BlockSpec((1, 1, CH, O), lambda b, n: (b, n, 0, 0)),
            pl.BlockSpec((1, O), lambda b, n: (0, 0)),
            pl.BlockSpec((1, O), lambda b, n: (0, 0)),
        ],
        out_shape=[
            jax.ShapeDtypeStruct((B, NB, CH, O), jnp.float32),
            jax.ShapeDtypeStruct((1, O), jnp.float32),
            jax.ShapeDtypeStruct((1, O), jnp.float32),
        ],
    )(nbr4, h4, w1t, w2t)


def _norm_body(ymax_ref, ssum_ref, ssq_ref, out_ref):
    bnk = float(B * N * K)
    mean = ssum_ref[...] / bnk
    e2 = ssq_ref[...] / bnk
    var = e2 - mean * mean
    sd = jnp.sqrt(var + 1e-5)
    for b in range(B):
        ym = (ymax_ref[b] - mean) / sd
        out_ref[b] = jnp.where(ym > 0, ym, 0.2 * ym)


def _norm(ymax, ssum, ssq):
    O = ymax.shape[2]
    return pl.pallas_call(
        _norm_body,
        out_shape=jax.ShapeDtypeStruct((B, N, O), jnp.float32),
    )(ymax, ssum, ssq)


def _final_body(h1_ref, h2_ref, h3_ref, h4_ref, wft_ref, bf_ref, out_ref):
    for b in range(B):
        cat = jnp.concatenate(
            [h1_ref[b], h2_ref[b], h3_ref[b], h4_ref[b]], axis=1)
        y = jnp.dot(cat, wft_ref[...], preferred_element_type=jnp.float32)
        y = y + bf_ref[...]
        out_ref[pl.ds(b, 1), :] = jnp.max(y, axis=0, keepdims=True)


def _final(hs, wft, bf2):
    return pl.pallas_call(
        _final_body,
        out_shape=jax.ShapeDtypeStruct((B, wft.shape[1]), jnp.float32),
    )(*hs, wft, bf2)


def kernel(x, W0, gamma0, beta0, W1, gamma1, beta1, W2, gamma2, beta2,
           W3, gamma3, beta3, Wf, bf):
    h = jnp.pad(x, ((0, 0), (0, 0), (0, 13)))
    offs = (jnp.arange(B, dtype=jnp.int32) * N)[:, None, None]

    hs = []
    for W in (W0, W1, W2, W3):
        O, C2 = W.shape
        C = C2 // 2
        w1 = W[:, :C]
        w2 = W[:, C:]
        w1t = jnp.transpose(w1)
        w2t = jnp.transpose(w2)
        if C == 3:
            w1t = jnp.pad(w1t, ((0, 13), (0, 0)))
            w2t = jnp.pad(w2t, ((0, 13), (0, 0)))
        Cp = w1t.shape[0]
        xx = jnp.sum(h * h, axis=2).reshape(B, 1, N)
        pd, tk = _pre(h, xx)
        nbr = _gather(Cp)(h.reshape(B * N, Cp), pd.reshape(-1),
                          tk.reshape(B * N, L))
        nbr4 = nbr.reshape(B, NB, CH * K, Cp)
        h4 = h.reshape(B, NB, CH, Cp)
        ymax, ssum, ssq = _conv(nbr4, h4, w1t, w2t)
        h = _norm(ymax.reshape(B, N, O), ssum, ssq)
        hs.append(h)

    wft = jnp.transpose(Wf)
    return _final(hs, wft, bf.reshape(1, -1))


# speedup vs baseline: 1.1510x; 1.1510x over previous
import functools

import jax
import jax.numpy as jnp
from jax import lax
from jax.experimental import pallas as pl
from jax.experimental.pallas import tpu as pltpu
from jax.experimental.pallas import tpu_sc as plsc

K = 40
B, N = 4, 1024
NC, NS, L = 2, 16, 16
NW = NC * NS
P = (B * N) // NW
PC = 4
RK = PC * K
NCH = P // PC
CH = 32
NB = N // CH


def _pre_body(h_ref, xx_ref, pd_ref, tk_ref):
    h = h_ref[0]
    g = lax.dot_general(h, h, (((1,), (1,)), ((), ())),
                        preferred_element_type=jnp.float32)
    pd = 2.0 * g - xx_ref[0]
    pd_ref[0] = pd
    s = lax.bitcast_convert_type(pd, jnp.int32)
    key = jnp.where(s < 0, s ^ jnp.int32(0x7FFFFFFF), s)

    def bit_step(i, t):
        bit = jnp.int32(1) << (jnp.int32(30) - i)
        cand = t + bit
        cnt = jnp.sum((key >= cand).astype(jnp.float32), axis=0, keepdims=True)
        return jnp.where(cnt >= float(K), cand, t)

    cnt0 = jnp.sum((key >= 0).astype(jnp.float32), axis=0, keepdims=True)
    t0 = jnp.where(cnt0 >= float(K), jnp.int32(0), jnp.int32(-2147483648))
    tk_ref[0] = lax.fori_loop(0, 31, bit_step, t0)


def _pre(h, xx):
    C = h.shape[2]
    return pl.pallas_call(
        _pre_body,
        grid=(B,),
        in_specs=[
            pl.BlockSpec((1, N, C), lambda b: (b, 0, 0)),
            pl.BlockSpec((1, N, 1), lambda b: (b, 0, 0)),
        ],
        out_specs=[
            pl.BlockSpec((1, N, N), lambda b: (b, 0, 0)),
            pl.BlockSpec((1, 1, N), lambda b: (b, 0, 0)),
        ],
        out_shape=[
            jax.ShapeDtypeStruct((B, N, N), jnp.float32),
            jax.ShapeDtypeStruct((B, 1, N), jnp.int32),
        ],
    )(h, xx)


def _make_gather(C):
    mesh = plsc.VectorSubcoreMesh(core_axis_name="c", subcore_axis_name="s",
                                  num_cores=NC, num_subcores=NS)

    @functools.partial(
        pl.kernel,
        out_type=jax.ShapeDtypeStruct((B * N * K, C), jnp.float32),
        mesh=mesh,
        compiler_params=pltpu.CompilerParams(use_tc_tiling_on_sc=False,
                                             needs_layout_passes=False),
        scratch_types=[
            pltpu.VMEM((P * K,), jnp.int32),
            pltpu.VMEM((RK, C), jnp.float32),
            pltpu.VMEM((RK, C), jnp.float32),
            pltpu.VMEM((N, L), jnp.float32),
            pltpu.VMEM((N, L), jnp.float32),
            pltpu.VMEM((P,), jnp.int32),
            pltpu.SemaphoreType.DMA,
            pltpu.SemaphoreType.DMA,
            pltpu.SemaphoreType.DMA,
            pltpu.SemaphoreType.DMA,
        ],
    )
    def gather_kernel(h_hbm, pd_hbm, tk_hbm, nbr_hbm,
                      idx_v, b0, b1, r0, r1, tk_v,
                      sem0, sem1, rs0, rs1):
        wid = lax.axis_index("s") * NC + lax.axis_index("c")
        base = wid * P * K
        pbase = wid * P
        bid = wid // (N // P)
        bn0 = bid * N
        col0 = pbase - bn0
        pltpu.sync_copy(tk_hbm.at[pl.ds(pbase, P)], tk_v)

        rbufs = (r0, r1)
        rsems = (rs0, rs1)
        NG = P // L

        def rfire(g, slot):
            pltpu.async_copy(
                pd_hbm.at[pl.ds(bn0, N), pl.ds(col0 + g * L, L)],
                rbufs[slot], rsems[slot])

        def rwait(slot):
            pltpu.make_async_copy(
                pd_hbm.at[pl.ds(bn0, N), pl.ds(col0, L)],
                rbufs[slot], rsems[slot]).wait()

        iota16 = lax.iota(jnp.int32, L)

        def select(g, slot):
            col = rbufs[slot]
            tvec = tk_v[pl.ds(g * L, L)]
            lanebase = (g * L + iota16) * K

            def jstep(j, run):
                s = plsc.bitcast(col[j], jnp.int32)
                keyv = jnp.where(s < 0, s ^ jnp.int32(0x7FFFFFFF), s)
                ge = keyv >= tvec
                gei = jnp.where(ge, jnp.int32(1), jnp.int32(0))
                okm = jnp.logical_and(ge, run < K)
                val = jnp.full((L,), jnp.int32(0)) + (bn0 + j)
                plsc.store_scatter(idx_v, (lanebase + run,), val, mask=okm)
                return run + gei

            lax.fori_loop(0, N, jstep, jnp.zeros((L,), jnp.int32))

        rfire(0, 0)

        def sel_step(q, _):
            g0 = 2 * q
            g1 = 2 * q + 1
            rfire(g1, 1)
            rwait(0)
            select(g0, 0)

            @pl.when(g1 + 1 < NG)
            def _():
                rfire(g1 + 1, 0)

            rwait(1)
            select(g1, 1)
            return 0

        lax.fori_loop(0, NG // 2, sel_step, 0)

        bufs = (b0, b1)
        sems = (sem0, sem1)

        def fire(c, slot):
            pltpu.async_copy(h_hbm.at[idx_v.at[pl.ds(c * RK, RK)]],
                             bufs[slot], sems[slot])

        def wait(slot):
            pltpu.make_async_copy(h_hbm.at[idx_v.at[pl.ds(0, RK)]],
                                  bufs[slot], sems[slot]).wait()

        def flush(c, slot):
            pltpu.sync_copy(bufs[slot], nbr_hbm.at[pl.ds(base + c * RK, RK)])

        fire(0, 0)

        def step(g, _):
            c0 = 2 * g
            c1 = 2 * g + 1
            fire(c1, 1)
            wait(0)
            flush(c0, 0)

            @pl.when(c1 + 1 < NCH)
            def _():
                fire(c1 + 1, 0)

            wait(1)
            flush(c1, 1)
            return 0

        lax.fori_loop(0, NCH // 2, step, 0)

    return gather_kernel


_GATHER_CACHE = {}


def _gather(C):
    if C not in _GATHER_CACHE:
        _GATHER_CACHE[C] = _make_gather(C)
    return _GATHER_CACHE[C]


def _conv_body(nbr_ref, h_ref, w1t_ref, w2t_ref, ymax_ref, ssum_ref, ssq_ref):
    b = pl.program_id(0)
    n = pl.program_id(1)
    ctr = h_ref[0, 0]
    nbr = nbr_ref[0, 0]
    C = ctr.shape[1]
    diff = (nbr.reshape(CH, K, C) - ctr[:, None, :]).reshape(CH * K, C)
    yd = jnp.dot(diff, w1t_ref[...], preferred_element_type=jnp.float32)
    yc = jnp.dot(ctr, w2t_ref[...], preferred_element_type=jnp.float32)
    O = yd.shape[1]
    y = yd.reshape(CH, K, O) + yc[:, None, :]
    ymax_ref[0, 0] = jnp.max(y, axis=1)
    y2 = y.reshape(CH * K, O)
    ps = jnp.sum(y2, axis=0, keepdims=True)
    pq = jnp.sum(y2 * y2, axis=0, keepdims=True)

    @pl.when(jnp.logical_and(b == 0, n == 0))
    def _():
        ssum_ref[...] = ps
        ssq_ref[...] = pq

    @pl.when(jnp.logical_or(b > 0, n > 0))
    def _():
        ssum_ref[...] = ssum_ref[...] + ps
        ssq_ref[...] = ssq_ref[...] + pq


def _conv(nbr4, h4, w1t, w2t):
    C = w1t.shape[0]
    O = w1t.shape[1]
    return pl.pallas_call(
        _conv_body,
        grid=(B, NB),
        in_specs=[
            pl.BlockSpec((1, 1, CH * K, C), lambda b, n: (b, n, 0, 0)),
            pl.BlockSpec((1, 1, CH, C), lambda b, n: (b, n, 0, 0)),
            pl.BlockSpec((C, O), lambda b, n: (0, 0)),
            pl.BlockSpec((C, O), lambda b, n: (0, 0)),
        ],
        out_specs=[
            pl.BlockSpec((1, 1, CH, O), lambda b, n: (b, n, 0, 0)),
            pl.BlockSpec((1, O), lambda b, n: (0, 0)),
            pl.BlockSpec((1, O), lambda b, n: (0, 0)),
        ],
        out_shape=[
            jax.ShapeDtypeStruct((B, NB, CH, O), jnp.float32),
            jax.ShapeDtypeStruct((1, O), jnp.float32),
            jax.ShapeDtypeStruct((1, O), jnp.float32),
        ],
    )(nbr4, h4, w1t, w2t)


def _norm_body(ymax_ref, ssum_ref, ssq_ref, out_ref):
    bnk = float(B * N * K)
    mean = ssum_ref[...] / bnk
    e2 = ssq_ref[...] / bnk
    var = e2 - mean * mean
    sd = jnp.sqrt(var + 1e-5)
    for b in range(B):
        ym = (ymax_ref[b] - mean) / sd
        out_ref[b] = jnp.where(ym > 0, ym, 0.2 * ym)


def _norm(ymax, ssum, ssq):
    O = ymax.shape[2]
    return pl.pallas_call(
        _norm_body,
        out_shape=jax.ShapeDtypeStruct((B, N, O), jnp.float32),
    )(ymax, ssum, ssq)


def _final_body(h1_ref, h2_ref, h3_ref, h4_ref, wft_ref, bf_ref, out_ref):
    for b in range(B):
        cat = jnp.concatenate(
            [h1_ref[b], h2_ref[b], h3_ref[b], h4_ref[b]], axis=1)
        y = jnp.dot(cat, wft_ref[...], preferred_element_type=jnp.float32)
        y = y + bf_ref[...]
        out_ref[pl.ds(b, 1), :] = jnp.max(y, axis=0, keepdims=True)


def _final(hs, wft, bf2):
    return pl.pallas_call(
        _final_body,
        out_shape=jax.ShapeDtypeStruct((B, wft.shape[1]), jnp.float32),
    )(*hs, wft, bf2)


def kernel(x, W0, gamma0, beta0, W1, gamma1, beta1, W2, gamma2, beta2,
           W3, gamma3, beta3, Wf, bf):
    h = jnp.pad(x, ((0, 0), (0, 0), (0, 13)))
    offs = (jnp.arange(B, dtype=jnp.int32) * N)[:, None, None]

    hs = []
    for W in (W0, W1, W2, W3):
        O, C2 = W.shape
        C = C2 // 2
        w1 = W[:, :C]
        w2 = W[:, C:]
        w1t = jnp.transpose(w1)
        w2t = jnp.transpose(w2)
        if C == 3:
            w1t = jnp.pad(w1t, ((0, 13), (0, 0)))
            w2t = jnp.pad(w2t, ((0, 13), (0, 0)))
        Cp = w1t.shape[0]
        xx = jnp.sum(h * h, axis=2).reshape(B, N, 1)
        pd, tk = _pre(h, xx)
        nbr = _gather(Cp)(h.reshape(B * N, Cp), pd.reshape(B * N, N),
                          tk.reshape(B * N))
        nbr4 = nbr.reshape(B, NB, CH * K, Cp)
        h4 = h.reshape(B, NB, CH, Cp)
        ymax, ssum, ssq = _conv(nbr4, h4, w1t, w2t)
        h = _norm(ymax.reshape(B, N, O), ssum, ssq)
        hs.append(h)

    wft = jnp.transpose(Wf)
    return _final(hs, wft, bf.reshape(1, -1))


# speedup vs baseline: 1.1575x; 1.0057x over previous
import functools

import jax
import jax.numpy as jnp
from jax import lax
from jax.experimental import pallas as pl
from jax.experimental.pallas import tpu as pltpu
from jax.experimental.pallas import tpu_sc as plsc

K = 40
B, N = 4, 1024
NC, NS, L = 2, 16, 16
NW = NC * NS
P = (B * N) // NW
PC = 8
RK = PC * K
NCH = P // PC
CH = 32
NB = N // CH


def _pre_body(h_ref, xx_ref, pd_ref, tk_ref):
    h = h_ref[0]
    g = lax.dot_general(h, h, (((1,), (1,)), ((), ())),
                        preferred_element_type=jnp.float32)
    pd = 2.0 * g - xx_ref[0]
    pd_ref[0] = pd
    s = lax.bitcast_convert_type(pd, jnp.int32)
    key = jnp.where(s < 0, s ^ jnp.int32(0x7FFFFFFF), s)

    def bit_step(i, t):
        bit = jnp.int32(1) << (jnp.int32(30) - i)
        cand = t + bit
        cnt = jnp.sum((key >= cand).astype(jnp.float32), axis=0, keepdims=True)
        return jnp.where(cnt >= float(K), cand, t)

    cnt0 = jnp.sum((key >= 0).astype(jnp.float32), axis=0, keepdims=True)
    t0 = jnp.where(cnt0 >= float(K), jnp.int32(0), jnp.int32(-2147483648))
    tk_ref[0] = lax.fori_loop(0, 31, bit_step, t0)


def _pre(h, xx):
    C = h.shape[2]
    return pl.pallas_call(
        _pre_body,
        grid=(B,),
        in_specs=[
            pl.BlockSpec((1, N, C), lambda b: (b, 0, 0)),
            pl.BlockSpec((1, N, 1), lambda b: (b, 0, 0)),
        ],
        out_specs=[
            pl.BlockSpec((1, N, N), lambda b: (b, 0, 0)),
            pl.BlockSpec((1, 1, N), lambda b: (b, 0, 0)),
        ],
        out_shape=[
            jax.ShapeDtypeStruct((B, N, N), jnp.float32),
            jax.ShapeDtypeStruct((B, 1, N), jnp.int32),
        ],
    )(h, xx)


def _make_gather(C):
    mesh = plsc.VectorSubcoreMesh(core_axis_name="c", subcore_axis_name="s",
                                  num_cores=NC, num_subcores=NS)

    @functools.partial(
        pl.kernel,
        out_type=jax.ShapeDtypeStruct((B * N * K, C), jnp.float32),
        mesh=mesh,
        compiler_params=pltpu.CompilerParams(use_tc_tiling_on_sc=False,
                                             needs_layout_passes=False),
        scratch_types=[
            pltpu.VMEM((P * K,), jnp.int32),
            pltpu.VMEM((RK, C), jnp.float32),
            pltpu.VMEM((RK, C), jnp.float32),
            pltpu.VMEM((N, L), jnp.float32),
            pltpu.VMEM((N, L), jnp.float32),
            pltpu.VMEM((P,), jnp.int32),
            pltpu.SemaphoreType.DMA,
            pltpu.SemaphoreType.DMA,
            pltpu.SemaphoreType.DMA,
            pltpu.SemaphoreType.DMA,
        ],
    )
    def gather_kernel(h_hbm, pd_hbm, tk_hbm, nbr_hbm,
                      idx_v, b0, b1, r0, r1, tk_v,
                      sem0, sem1, rs0, rs1):
        wid = lax.axis_index("s") * NC + lax.axis_index("c")
        base = wid * P * K
        pbase = wid * P
        bid = wid // (N // P)
        bn0 = bid * N
        col0 = pbase - bn0
        pltpu.sync_copy(tk_hbm.at[pl.ds(pbase, P)], tk_v)

        rbufs = (r0, r1)
        rsems = (rs0, rs1)
        NG = P // L

        def rfire(g, slot):
            pltpu.async_copy(
                pd_hbm.at[pl.ds(bn0, N), pl.ds(col0 + g * L, L)],
                rbufs[slot], rsems[slot])

        def rwait(slot):
            pltpu.make_async_copy(
                pd_hbm.at[pl.ds(bn0, N), pl.ds(col0, L)],
                rbufs[slot], rsems[slot]).wait()

        iota16 = lax.iota(jnp.int32, L)

        def select(g, slot):
            col = rbufs[slot]
            tvec = tk_v[pl.ds(g * L, L)]
            lanebase = (g * L + iota16) * K

            def one(j, run):
                s = plsc.bitcast(col[j], jnp.int32)
                keyv = jnp.where(s < 0, s ^ jnp.int32(0x7FFFFFFF), s)
                ge = keyv >= tvec
                gei = jnp.where(ge, jnp.int32(1), jnp.int32(0))
                okm = jnp.logical_and(ge, run < K)
                val = jnp.full((L,), jnp.int32(0)) + (bn0 + j)
                plsc.store_scatter(idx_v, (lanebase + run,), val, mask=okm)
                return run + gei

            def jstep(i, run):
                j = 4 * i
                return one(j + 3, one(j + 2, one(j + 1, one(j, run))))

            lax.fori_loop(0, N // 4, jstep, jnp.zeros((L,), jnp.int32))

        rfire(0, 0)

        def sel_step(q, _):
            g0 = 2 * q
            g1 = 2 * q + 1
            rfire(g1, 1)
            rwait(0)
            select(g0, 0)

            @pl.when(g1 + 1 < NG)
            def _():
                rfire(g1 + 1, 0)

            rwait(1)
            select(g1, 1)
            return 0

        lax.fori_loop(0, NG // 2, sel_step, 0)

        bufs = (b0, b1)
        sems = (sem0, sem1)

        def fire(c, slot):
            pltpu.async_copy(h_hbm.at[idx_v.at[pl.ds(c * RK, RK)]],
                             bufs[slot], sems[slot])

        def wait(slot):
            pltpu.make_async_copy(h_hbm.at[idx_v.at[pl.ds(0, RK)]],
                                  bufs[slot], sems[slot]).wait()

        def flush(c, slot):
            pltpu.sync_copy(bufs[slot], nbr_hbm.at[pl.ds(base + c * RK, RK)])

        fire(0, 0)

        def step(g, _):
            c0 = 2 * g
            c1 = 2 * g + 1
            fire(c1, 1)
            wait(0)
            flush(c0, 0)

            @pl.when(c1 + 1 < NCH)
            def _():
                fire(c1 + 1, 0)

            wait(1)
            flush(c1, 1)
            return 0

        lax.fori_loop(0, NCH // 2, step, 0)

    return gather_kernel


_GATHER_CACHE = {}


def _gather(C):
    if C not in _GATHER_CACHE:
        _GATHER_CACHE[C] = _make_gather(C)
    return _GATHER_CACHE[C]


def _conv_body(nbr_ref, h_ref, w1t_ref, w2t_ref, ymax_ref, ssum_ref, ssq_ref):
    b = pl.program_id(0)
    n = pl.program_id(1)
    ctr = h_ref[0, 0]
    nbr = nbr_ref[0, 0]
    C = ctr.shape[1]
    diff = (nbr.reshape(CH, K, C) - ctr[:, None, :]).reshape(CH * K, C)
    yd = jnp.dot(diff, w1t_ref[...], preferred_element_type=jnp.float32)
    yc = jnp.dot(ctr, w2t_ref[...], preferred_element_type=jnp.float32)
    O = yd.shape[1]
    y = yd.reshape(CH, K, O) + yc[:, None, :]
    ymax_ref[0, 0] = jnp.max(y, axis=1)
    y2 = y.reshape(CH * K, O)
    ps = jnp.sum(y2, axis=0, keepdims=True)
    pq = jnp.sum(y2 * y2, axis=0, keepdims=True)

    @pl.when(jnp.logical_and(b == 0, n == 0))
    def _():
        ssum_ref[...] = ps
        ssq_ref[...] = pq

    @pl.when(jnp.logical_or(b > 0, n > 0))
    def _():
        ssum_ref[...] = ssum_ref[...] + ps
        ssq_ref[...] = ssq_ref[...] + pq


def _conv(nbr4, h4, w1t, w2t):
    C = w1t.shape[0]
    O = w1t.shape[1]
    return pl.pallas_call(
        _conv_body,
        grid=(B, NB),
        in_specs=[
            pl.BlockSpec((1, 1, CH * K, C), lambda b, n: (b, n, 0, 0)),
            pl.BlockSpec((1, 1, CH, C), lambda b, n: (b, n, 0, 0)),
            pl.BlockSpec((C, O), lambda b, n: (0, 0)),
            pl.BlockSpec((C, O), lambda b, n: (0, 0)),
        ],
        out_specs=[
            pl.BlockSpec((1, 1, CH, O), lambda b, n: (b, n, 0, 0)),
            pl.BlockSpec((1, O), lambda b, n: (0, 0)),
            pl.BlockSpec((1, O), lambda b, n: (0, 0)),
        ],
        out_shape=[
            jax.ShapeDtypeStruct((B, NB, CH, O), jnp.float32),
            jax.ShapeDtypeStruct((1, O), jnp.float32),
            jax.ShapeDtypeStruct((1, O), jnp.float32),
        ],
    )(nbr4, h4, w1t, w2t)


def _norm_body(ymax_ref, ssum_ref, ssq_ref, out_ref):
    bnk = float(B * N * K)
    mean = ssum_ref[...] / bnk
    e2 = ssq_ref[...] / bnk
    var = e2 - mean * mean
    sd = jnp.sqrt(var + 1e-5)
    for b in range(B):
        ym = (ymax_ref[b] - mean) / sd
        out_ref[b] = jnp.where(ym > 0, ym, 0.2 * ym)


def _norm(ymax, ssum, ssq):
    O = ymax.shape[2]
    return pl.pallas_call(
        _norm_body,
        out_shape=jax.ShapeDtypeStruct((B, N, O), jnp.float32),
    )(ymax, ssum, ssq)


def _final_body(h1_ref, h2_ref, h3_ref, h4_ref, wft_ref, bf_ref, out_ref):
    for b in range(B):
        cat = jnp.concatenate(
            [h1_ref[b], h2_ref[b], h3_ref[b], h4_ref[b]], axis=1)
        y = jnp.dot(cat, wft_ref[...], preferred_element_type=jnp.float32)
        y = y + bf_ref[...]
        out_ref[pl.ds(b, 1), :] = jnp.max(y, axis=0, keepdims=True)


def _final(hs, wft, bf2):
    return pl.pallas_call(
        _final_body,
        out_shape=jax.ShapeDtypeStruct((B, wft.shape[1]), jnp.float32),
    )(*hs, wft, bf2)


def kernel(x, W0, gamma0, beta0, W1, gamma1, beta1, W2, gamma2, beta2,
           W3, gamma3, beta3, Wf, bf):
    h = jnp.pad(x, ((0, 0), (0, 0), (0, 13)))
    offs = (jnp.arange(B, dtype=jnp.int32) * N)[:, None, None]

    hs = []
    for W in (W0, W1, W2, W3):
        O, C2 = W.shape
        C = C2 // 2
        w1 = W[:, :C]
        w2 = W[:, C:]
        w1t = jnp.transpose(w1)
        w2t = jnp.transpose(w2)
        if C == 3:
            w1t = jnp.pad(w1t, ((0, 13), (0, 0)))
            w2t = jnp.pad(w2t, ((0, 13), (0, 0)))
        Cp = w1t.shape[0]
        xx = jnp.sum(h * h, axis=2).reshape(B, N, 1)
        pd, tk = _pre(h, xx)
        nbr = _gather(Cp)(h.reshape(B * N, Cp), pd.reshape(B * N, N),
                          tk.reshape(B * N))
        nbr4 = nbr.reshape(B, NB, CH * K, Cp)
        h4 = h.reshape(B, NB, CH, Cp)
        ymax, ssum, ssq = _conv(nbr4, h4, w1t, w2t)
        h = _norm(ymax.reshape(B, N, O), ssum, ssq)
        hs.append(h)

    wft = jnp.transpose(Wf)
    return _final(hs, wft, bf.reshape(1, -1))
